# baseline (speedup 1.0000x reference)
import numpy as np
import jax
import jax.numpy as jnp
from jax.experimental import pallas as pl

_B, _Q, _N = 64, 8, 32768
_R = 8
_K = 8
_C = 256
_L = 128

_G = np.asarray(
    jax.random.gumbel(jax.random.key(1), (_B, _Q, _N), dtype=jnp.float32)
).reshape(_B * _Q, _C, _L)

_NEG = -np.inf


def _body(s_ref, g_ref, o_ref):
    x = s_ref[...] + g_ref[...]
    cm = jnp.max(x, axis=2)
    ci = jax.lax.broadcasted_iota(jnp.int32, (_R, _C), 1)
    cidx = []
    for _ in range(_K):
        m = jnp.max(cm, axis=1, keepdims=True)
        idx = jnp.min(jnp.where(cm == m, ci, jnp.int32(_C)),
                      axis=1, keepdims=True)
        cidx.append(idx)
        cm = jnp.where(ci == idx, _NEG, cm)
    cols = _R * _C
    cc = jax.lax.broadcasted_iota(jnp.int32, (_R, cols), 1)
    col_r = jax.lax.shift_right_logical(cc, 8)
    col_c = jnp.bitwise_and(cc, jnp.int32(_C - 1))
    row_r = jax.lax.broadcasted_iota(jnp.int32, (_R, cols), 0)
    s_mat = jnp.concatenate(
        [jnp.logical_and(col_r == row_r, col_c == cidx[j]).astype(jnp.float32)
         for j in range(_K)], axis=0)
    x_flat = x.reshape(cols, _L)
    y = jax.lax.dot_general(s_mat, x_flat, (((1,), (0,)), ((), ())),
                            preferred_element_type=jnp.float32,
                            precision=jax.lax.Precision.HIGHEST)
    y3 = y.reshape(_K, _R, _L)
    li = jax.lax.broadcasted_iota(jnp.int32, (1, _R, _L), 2)
    gidx = jnp.concatenate(
        [cidx[j][None] * _L + li for j in range(_K)], axis=0)
    hot = jnp.zeros((_K, _R, _L), jnp.float32)
    yv = y3
    for _ in range(_K):
        m = jnp.max(yv, axis=(0, 2), keepdims=True)
        g_at = jnp.min(jnp.where(yv == m, gidx, jnp.int32(_N)),
                       axis=(0, 2), keepdims=True)
        hit = gidx == g_at
        hot = jnp.where(hit, 1.0, hot)
        yv = jnp.where(hit, _NEG, yv)
    hot_flat = hot.reshape(_K * _R, _L)
    o_flat = jax.lax.dot_general(s_mat, hot_flat, (((0,), (0,)), ((), ())),
                                 preferred_element_type=jnp.float32,
                            precision=jax.lax.Precision.HIGHEST)
    o_ref[...] = o_flat.reshape(_R, _C, _L)


def kernel(scores):
    s3 = scores.reshape(_B * _Q, _C, _L)
    out = pl.pallas_call(
        _body,
        grid=(_B * _Q // _R,),
        in_specs=[
            pl.BlockSpec((_R, _C, _L), lambda i: (i, 0, 0)),
            pl.BlockSpec((_R, _C, _L), lambda i: (i, 0, 0)),
        ],
        out_specs=pl.BlockSpec((_R, _C, _L), lambda i: (i, 0, 0)),
        out_shape=jax.ShapeDtypeStruct((_B * _Q, _C, _L), jnp.float32),
    )(s3, jnp.asarray(_G))
    return out.reshape(_B, _Q, _N)


# speedup vs baseline: 1.0925x; 1.0925x over previous
import numpy as np
import jax
import jax.numpy as jnp
from jax.experimental import pallas as pl

_B, _Q, _N = 64, 8, 32768
_R = 8
_K = 8
_C = 256
_L = 128

_G = np.asarray(
    jax.random.gumbel(jax.random.key(1), (_B, _Q, _N), dtype=jnp.float32)
).reshape(_B * _Q, _C, _L)

_NEG = -np.inf


def _body(s_ref, g_ref, o_ref):
    x = s_ref[...] + g_ref[...]
    cm = jnp.max(x, axis=2)
    ci = jax.lax.broadcasted_iota(jnp.int32, (_R, _C), 1)
    cidx = []
    for _ in range(_K):
        m = jnp.max(cm, axis=1, keepdims=True)
        idx = jnp.min(jnp.where(cm == m, ci, jnp.int32(_C)),
                      axis=1, keepdims=True)
        cidx.append(idx)
        cm = jnp.where(ci == idx, _NEG, cm)
    cols = _R * _C
    cc = jax.lax.broadcasted_iota(jnp.int32, (_R, cols), 1)
    col_r = jax.lax.shift_right_logical(cc, 8)
    col_c = jnp.bitwise_and(cc, jnp.int32(_C - 1))
    row_r = jax.lax.broadcasted_iota(jnp.int32, (_R, cols), 0)
    s_mat = jnp.concatenate(
        [jnp.logical_and(col_r == row_r, col_c == cidx[j]).astype(jnp.float32)
         for j in range(_K)], axis=0)
    x_flat = x.reshape(cols, _L)
    y = jax.lax.dot_general(s_mat, x_flat, (((1,), (0,)), ((), ())),
                            preferred_element_type=jnp.float32,
                            precision=jax.lax.Precision.HIGHEST)
    y3 = y.reshape(_K, _R, _L)
    li = jax.lax.broadcasted_iota(jnp.int32, (1, _R, _L), 2)
    gidx = jnp.concatenate(
        [cidx[j][None] * _L + li for j in range(_K)], axis=0)
    hot = jnp.zeros((_K, _R, _L), jnp.float32)
    yv = y3
    for _ in range(_K):
        m = jnp.max(yv, axis=(0, 2), keepdims=True)
        g_at = jnp.min(jnp.where(yv == m, gidx, jnp.int32(_N)),
                       axis=(0, 2), keepdims=True)
        hit = gidx == g_at
        hot = jnp.where(hit, 1.0, hot)
        yv = jnp.where(hit, _NEG, yv)
    hot_flat = hot.reshape(_K * _R, _L)
    o_flat = jax.lax.dot_general(s_mat, hot_flat, (((0,), (0,)), ((), ())),
                                 preferred_element_type=jnp.float32)
    o_ref[...] = o_flat.reshape(_R, _C, _L)


def kernel(scores):
    s3 = scores.reshape(_B * _Q, _C, _L)
    out = pl.pallas_call(
        _body,
        grid=(_B * _Q // _R,),
        in_specs=[
            pl.BlockSpec((_R, _C, _L), lambda i: (i, 0, 0)),
            pl.BlockSpec((_R, _C, _L), lambda i: (i, 0, 0)),
        ],
        out_specs=pl.BlockSpec((_R, _C, _L), lambda i: (i, 0, 0)),
        out_shape=jax.ShapeDtypeStruct((_B * _Q, _C, _L), jnp.float32),
    )(s3, jnp.asarray(_G))
    return out.reshape(_B, _Q, _N)


# speedup vs baseline: 2.4770x; 2.2672x over previous
import numpy as np
import jax
import jax.numpy as jnp
from jax.experimental import pallas as pl

_B, _Q, _N = 64, 8, 32768
_R = 8
_K = 8

_G = np.asarray(
    jax.random.gumbel(jax.random.key(1), (_B, _Q, _N), dtype=jnp.float32)
).reshape(_B * _Q, _N)

_NEG = -np.inf


def _body(s_ref, g_ref, o_ref):
    x0 = s_ref[...] + g_ref[...]
    x = x0
    for _ in range(_K):
        m = jnp.max(x, axis=1, keepdims=True)
        x = jnp.where(x == m, _NEG, x)
    sel = x == _NEG
    cnt = jnp.sum(jnp.where(sel, 1.0, 0.0), axis=1, keepdims=True)
    o_ref[...] = jnp.where(sel, 1.0, 0.0)
    bad = jnp.logical_or(jnp.max(cnt) != 8.0, jnp.min(cnt) != 8.0)

    @pl.when(bad)
    def _fallback():
        iota = jax.lax.broadcasted_iota(jnp.int32, x0.shape, 1)
        xf = x0
        acc = jnp.zeros_like(x0)
        for _ in range(_K):
            m = jnp.max(xf, axis=1, keepdims=True)
            idx = jnp.min(jnp.where(xf == m, iota, jnp.int32(_N)),
                          axis=1, keepdims=True)
            hit = iota == idx
            acc = jnp.where(hit, 1.0, acc)
            xf = jnp.where(hit, _NEG, xf)
        o_ref[...] = acc


def kernel(scores):
    s2 = scores.reshape(_B * _Q, _N)
    out = pl.pallas_call(
        _body,
        grid=(_B * _Q // _R,),
        in_specs=[
            pl.BlockSpec((_R, _N), lambda i: (i, 0)),
            pl.BlockSpec((_R, _N), lambda i: (i, 0)),
        ],
        out_specs=pl.BlockSpec((_R, _N), lambda i: (i, 0)),
        out_shape=jax.ShapeDtypeStruct((_B * _Q, _N), jnp.float32),
    )(s2, jnp.asarray(_G))
    return out.reshape(_B, _Q, _N)
